# Initial kernel scaffold; baseline (speedup 1.0000x reference)
import jax
import jax.numpy as jnp
from jax.experimental import pallas as pl


def kernel(x, w1, w2):
    raise NotImplementedError("write your pallas kernel here")


# speedup vs baseline: 1.2017x; 1.2017x over previous
import functools

import jax
import jax.numpy as jnp
from jax.experimental import pallas as pl
from jax.experimental.pallas import tpu as pltpu


def _sigmoid(z):
    return 0.5 * jnp.tanh(0.5 * z) + 0.5


def _mlp_kernel(x_ref, w1_ref, w2_ref, o_ref):
    xb = x_ref[...].astype(jnp.bfloat16)
    h = jax.lax.dot_general(
        xb, w1_ref[...], (((1,), (1,)), ((), ())),
        preferred_element_type=jnp.float32)
    hb = _sigmoid(h).astype(jnp.bfloat16)
    y = jax.lax.dot_general(
        hb, w2_ref[...], (((1,), (1,)), ((), ())),
        preferred_element_type=jnp.float32)
    o_ref[...] = _sigmoid(y)


@functools.partial(jax.jit, static_argnames=("batch_block",))
def _mlp_forward(x, w1, w2, batch_block=512):
    batch, input_size = x.shape
    hidden_size, _ = w1.shape
    output_size, _ = w2.shape

    w1b = w1.astype(jnp.bfloat16)
    w2b = w2.astype(jnp.bfloat16)

    n_blocks = pl.cdiv(batch, batch_block)
    padded_batch = n_blocks * batch_block
    if padded_batch != batch:
        x = jnp.pad(x, ((0, padded_batch - batch), (0, 0)))

    out = pl.pallas_call(
        _mlp_kernel,
        out_shape=jax.ShapeDtypeStruct((padded_batch, output_size), jnp.float32),
        grid=(n_blocks,),
        in_specs=[
            pl.BlockSpec((batch_block, input_size), lambda i: (i, 0)),
            pl.BlockSpec((hidden_size, input_size), lambda i: (0, 0)),
            pl.BlockSpec((output_size, hidden_size), lambda i: (0, 0)),
        ],
        out_specs=pl.BlockSpec((batch_block, output_size), lambda i: (i, 0)),
        compiler_params=pltpu.CompilerParams(
            dimension_semantics=("parallel",),
        ),
    )(x, w1b, w2b)

    if padded_batch != batch:
        out = out[:batch]
    return out


def kernel(x, w1, w2):
    return _mlp_forward(x, w1, w2)
